# baseline (speedup 1.0000x reference)
import functools

import jax
import jax.numpy as jnp
from jax import lax
from jax.experimental import pallas as pl
from jax.experimental.pallas import tpu as pltpu
from jax.experimental.pallas import tpu_sc as plsc

N = 10000
R = 16
E = 320000
H0 = 128
H1 = 16

NC = 2
NS = 16
L = 16

SUB = 80
EROWS = E // SUB
RPB = 25
BLKE = RPB * SUB
AROWS = EROWS // NS
ABLK = AROWS // RPB
BROWS = EROWS // (NC * NS)
BBLK = BROWS // RPB
CSL = (R * N) // NS
OSL = N // NS

_GDN = lax.GatherDimensionNumbers(
    offset_dims=(), collapsed_slice_dims=(0,), start_index_map=(0,))


def _mm_body(n_ref, w_ref, o_ref):
    o_ref[...] = jnp.dot(n_ref[...], w_ref[...],
                         preferred_element_type=jnp.float32)


_BM = 2000
_mm = pl.pallas_call(
    _mm_body,
    grid=(N // _BM,),
    in_specs=[
        pl.BlockSpec((_BM, H0), lambda i: (i, 0)),
        pl.BlockSpec((H0, R * H1), lambda i: (0, 0)),
    ],
    out_specs=pl.BlockSpec((_BM, R * H1), lambda i: (i, 0)),
    out_shape=jax.ShapeDtypeStruct((N, R * H1), jnp.float32),
)


def _comb_body(p_ref, b_ref, o_ref):
    o_ref[...] = p_ref[0] + p_ref[1] + b_ref[...]


_comb = pl.pallas_call(
    _comb_body,
    in_specs=[
        pl.BlockSpec((NC, (N * H1) // 128, 128), lambda: (0, 0, 0)),
        pl.BlockSpec((1, 128), lambda: (0, 0)),
    ],
    out_specs=pl.BlockSpec(((N * H1) // 128, 128), lambda: (0, 0)),
    out_shape=jax.ShapeDtypeStruct(((N * H1) // 128, 128), jnp.float32),
)


def _sc_body(s2, p2, o2, nw2, part,
             counts_sp, out_sp, pb, qb, sb, keyb, valb, ones, rows, sem):
    cid = lax.axis_index("c")
    sid = lax.axis_index("s")

    with jax.named_scope("ph_init"):
        for k in range(SUB // L):
            ones[pl.ds(k * L, L)] = jnp.ones((L,), jnp.float32)

        def zf(i, _):
            valb[pl.ds(i * L, L)] = jnp.zeros((L,), jnp.float32)
            return 0

        lax.fori_loop(0, BLKE // L, zf, 0)

        def zr(i, _):
            rows[i] = jnp.zeros((L,), jnp.float32)
            return 0

        lax.fori_loop(0, OSL, zr, 0)

        for t in range(CSL // BLKE):
            pltpu.sync_copy(valb,
                            counts_sp.at[pl.ds(sid * CSL + t * BLKE, BLKE)])
        pltpu.sync_copy(rows.at[pl.ds(0, OSL)],
                        out_sp.at[pl.ds(sid * OSL, OSL)])
        plsc.subcore_barrier()

    arow0 = sid * AROWS

    def pa(b, _):
        r0 = arow0 + b * RPB
        pltpu.sync_copy(p2.at[pl.ds(r0, RPB)], pb)
        pltpu.sync_copy(s2.at[pl.ds(r0, RPB)], sb)

        def mk(j, _):
            for k in range(SUB // L):
                sl = pl.ds(k * L, L)
                keyb[j, sl] = pb[j, sl] * N + sb[j, sl]
            return 0

        lax.fori_loop(0, RPB, mk, 0)

        def scat(j, _):
            pltpu.sync_copy(ones, counts_sp.at[keyb.at[j]], add=True)
            return 0

        lax.fori_loop(0, RPB, scat, 0)
        return 0

    with jax.named_scope("ph_count"):
        lax.fori_loop(0, ABLK, pa, 0)
        plsc.subcore_barrier()

    brow0 = cid * (EROWS // NC) + sid * BROWS

    def pb_loop(g, _):
        r0 = brow0 + g * RPB
        _s1 = jax.named_scope("pb_stage"); _s1.__enter__()
        pltpu.sync_copy(p2.at[pl.ds(r0, RPB)], pb)
        pltpu.sync_copy(o2.at[pl.ds(r0, RPB)], qb)
        pltpu.sync_copy(s2.at[pl.ds(r0, RPB)], sb)

        def mk(j, _):
            for k in range(SUB // L):
                sl = pl.ds(k * L, L)
                keyb[j, sl] = pb[j, sl] * N + sb[j, sl]
            return 0

        lax.fori_loop(0, RPB, mk, 0)

        def gc(j, _):
            pltpu.sync_copy(counts_sp.at[keyb.at[j]],
                            valb.at[pl.ds(j * SUB, SUB)])
            return 0

        lax.fori_loop(0, RPB, gc, 0)

        def inv(i, _):
            sl = pl.ds(i * L, L)
            valb[sl] = 1.0 / valb[sl]
            return 0

        lax.fori_loop(0, BLKE // L, inv, 0)

        def mg(j, _):
            for k in range(SUB // L):
                sl = pl.ds(k * L, L)
                keyb[j, sl] = qb[j, sl] * H1 + pb[j, sl]
            return 0

        lax.fori_loop(0, RPB, mg, 0)
        _s1.__exit__(None, None, None)
        _s2 = jax.named_scope("pb_gather"); _s2.__enter__()

        def gr(q, _):
            descs = []
            for k in range(5):
                j = q * 5 + k
                descs.append(pltpu.async_copy(
                    nw2.at[keyb.at[j]], rows.at[pl.ds(j * SUB, SUB)], sem))
            for dsc in descs:
                dsc.wait()
            return 0

        lax.fori_loop(0, RPB // 5, gr, 0)
        _s2.__exit__(None, None, None)
        _s3 = jax.named_scope("pb_scale"); _s3.__enter__()

        def sc_(c, _):
            vch = valb[pl.ds(c * L, L)]
            base = c * L
            for j in range(L):
                v = lax.gather(
                    vch, jnp.full((L, 1), j, jnp.int32), _GDN,
                    slice_sizes=(1,),
                    mode=lax.GatherScatterMode.PROMISE_IN_BOUNDS)
                rows[base + j] = rows[base + j] * v
            return 0

        lax.fori_loop(0, BLKE // L, sc_, 0)
        _s3.__exit__(None, None, None)
        _s4 = jax.named_scope("pb_scatter"); _s4.__enter__()

        def sa(j, _):
            pltpu.sync_copy(rows.at[pl.ds(j * SUB, SUB)],
                            out_sp.at[sb.at[j]], add=True)
            return 0

        lax.fori_loop(0, RPB, sa, 0)
        _s4.__exit__(None, None, None)
        return 0

    lax.fori_loop(0, BBLK, pb_loop, 0)
    plsc.subcore_barrier()

    with jax.named_scope("ph_writeout"):
        o0 = sid * OSL
        pltpu.sync_copy(out_sp.at[pl.ds(o0, OSL)], rows.at[pl.ds(0, OSL)])
        pltpu.sync_copy(rows.at[pl.ds(0, OSL)],
                        part.at[cid, pl.ds(o0, OSL)])


@functools.cache
def _sc_kernel():
    mesh = plsc.VectorSubcoreMesh(core_axis_name="c", subcore_axis_name="s")
    return pl.kernel(
        _sc_body,
        out_type=jax.ShapeDtypeStruct((NC, N, H1), jnp.float32),
        mesh=mesh,
        compiler_params=pltpu.CompilerParams(use_tc_tiling_on_sc=False),
        scratch_types=[
            pltpu.VMEM_SHARED((R * N,), jnp.float32),
            pltpu.VMEM_SHARED((N, H1), jnp.float32),
            pltpu.VMEM((RPB, SUB), jnp.int32),
            pltpu.VMEM((RPB, SUB), jnp.int32),
            pltpu.VMEM((RPB, SUB), jnp.int32),
            pltpu.VMEM((RPB, SUB), jnp.int32),
            pltpu.VMEM((BLKE,), jnp.float32),
            pltpu.VMEM((SUB,), jnp.float32),
            pltpu.VMEM((BLKE, H1), jnp.float32),
            pltpu.SemaphoreType.DMA,
        ],
    )


def kernel(nodes, triples, weights, bias):
    s = triples[:, 0].reshape(EROWS, SUB)
    p = triples[:, 1].reshape(EROWS, SUB)
    o = triples[:, 2].reshape(EROWS, SUB)
    w_all = jnp.transpose(weights, (1, 0, 2)).reshape(H0, R * H1)
    nw2 = _mm(nodes, w_all).reshape(N * R, H1)
    part = _sc_kernel()(s, p, o, nw2)
    bias_t = jnp.tile(bias, H0 // H1).reshape(1, 128)
    out2 = _comb(part.reshape(NC, (N * H1) // 128, 128), bias_t)
    return out2.reshape(N, H1)


# speedup vs baseline: 1.4593x; 1.4593x over previous
import functools

import jax
import jax.numpy as jnp
from jax import lax
from jax.experimental import pallas as pl
from jax.experimental.pallas import tpu as pltpu
from jax.experimental.pallas import tpu_sc as plsc

N = 10000
R = 16
E = 320000
H0 = 128
H1 = 16

NC = 2
NS = 16
L = 16

SUB = 80
EROWS = E // SUB
RPB = 25
BLKE = RPB * SUB
AROWS = EROWS // NS
ABLK = AROWS // RPB
BROWS = EROWS // (NC * NS)
BBLK = BROWS // RPB
CSL = (R * N) // NS
OSL = N // NS

_GDN = lax.GatherDimensionNumbers(
    offset_dims=(), collapsed_slice_dims=(0,), start_index_map=(0,))


def _mm_body(n_ref, w_ref, o_ref):
    o_ref[...] = jnp.dot(n_ref[...], w_ref[...],
                         preferred_element_type=jnp.float32)


_BM = 2000
_mm = pl.pallas_call(
    _mm_body,
    grid=(N // _BM,),
    in_specs=[
        pl.BlockSpec((_BM, H0), lambda i: (i, 0)),
        pl.BlockSpec((H0, R * H1), lambda i: (0, 0)),
    ],
    out_specs=pl.BlockSpec((_BM, R * H1), lambda i: (i, 0)),
    out_shape=jax.ShapeDtypeStruct((N, R * H1), jnp.float32),
)


def _comb_body(p_ref, b_ref, o_ref):
    o_ref[...] = p_ref[0] + p_ref[1] + b_ref[...]


_comb = pl.pallas_call(
    _comb_body,
    in_specs=[
        pl.BlockSpec((NC, (N * H1) // 128, 128), lambda: (0, 0, 0)),
        pl.BlockSpec((1, 128), lambda: (0, 0)),
    ],
    out_specs=pl.BlockSpec(((N * H1) // 128, 128), lambda: (0, 0)),
    out_shape=jax.ShapeDtypeStruct(((N * H1) // 128, 128), jnp.float32),
)


def _sc_body(s2, p2, o2, nw2, part,
             counts_sp, out_sp, pb, qb, sb, keyb, gidxb, valb, ones,
             rows, sem, sem2, sem3):
    cid = lax.axis_index("c")
    sid = lax.axis_index("s")

    with jax.named_scope("ph_init"):
        for k in range(SUB // L):
            ones[pl.ds(k * L, L)] = jnp.ones((L,), jnp.float32)

        def zf(i, _):
            valb[pl.ds(i * L, L)] = jnp.zeros((L,), jnp.float32)
            return 0

        lax.fori_loop(0, BLKE // L, zf, 0)

        def zr(i, _):
            rows[i] = jnp.zeros((L,), jnp.float32)
            return 0

        lax.fori_loop(0, OSL, zr, 0)

        for t in range(CSL // BLKE):
            pltpu.sync_copy(valb,
                            counts_sp.at[pl.ds(sid * CSL + t * BLKE, BLKE)])
        pltpu.sync_copy(rows.at[pl.ds(0, OSL)],
                        out_sp.at[pl.ds(sid * OSL, OSL)])
        plsc.subcore_barrier()

    arow0 = sid * AROWS

    def pa(b, _):
        r0 = arow0 + b * RPB
        d1 = pltpu.async_copy(p2.at[pl.ds(r0, RPB)], pb, sem)
        d2 = pltpu.async_copy(s2.at[pl.ds(r0, RPB)], sb, sem)
        d1.wait(); d2.wait()

        def mk(j, _):
            for k in range(SUB // L):
                sl = pl.ds(k * L, L)
                keyb[j, sl] = pb[j, sl] * N + sb[j, sl]
            return 0

        lax.fori_loop(0, RPB, mk, 0, unroll=5)

        dsc = [pltpu.async_copy(ones, counts_sp.at[keyb.at[j]], sem3,
                                add=True)
               for j in range(RPB)]
        for d in dsc:
            d.wait()
        return 0

    with jax.named_scope("ph_count"):
        lax.fori_loop(0, ABLK, pa, 0)
        plsc.subcore_barrier()

    brow0 = cid * (EROWS // NC) + sid * BROWS

    def pb_loop(g, _):
        r0 = brow0 + g * RPB
        _s1 = jax.named_scope("pb_stage"); _s1.__enter__()
        d1 = pltpu.async_copy(p2.at[pl.ds(r0, RPB)], pb, sem)
        d2 = pltpu.async_copy(o2.at[pl.ds(r0, RPB)], qb, sem)
        d3 = pltpu.async_copy(s2.at[pl.ds(r0, RPB)], sb, sem)
        d1.wait(); d2.wait(); d3.wait()

        def mk(j, _):
            for k in range(SUB // L):
                sl = pl.ds(k * L, L)
                pv = pb[j, sl]
                keyb[j, sl] = pv * N + sb[j, sl]
                gidxb[j, sl] = qb[j, sl] * H1 + pv
            return 0

        lax.fori_loop(0, RPB, mk, 0, unroll=5)
        _s1.__exit__(None, None, None)
        _s2 = jax.named_scope("pb_gather"); _s2.__enter__()

        dc = [pltpu.async_copy(counts_sp.at[keyb.at[j]],
                               valb.at[pl.ds(j * SUB, SUB)], sem)
              for j in range(RPB)]
        dr = [pltpu.async_copy(nw2.at[gidxb.at[j]],
                               rows.at[pl.ds(j * SUB, SUB)], sem2)
              for j in range(RPB)]
        for d in dc:
            d.wait()

        def inv(i, _):
            sl = pl.ds(i * L, L)
            valb[sl] = 1.0 / valb[sl]
            return 0

        lax.fori_loop(0, BLKE // L, inv, 0, unroll=5)
        for d in dr:
            d.wait()
        _s2.__exit__(None, None, None)
        _s3 = jax.named_scope("pb_scale"); _s3.__enter__()

        def sc_(c, _):
            vch = valb[pl.ds(c * L, L)]
            base = c * L
            for j in range(L):
                v = lax.gather(
                    vch, jnp.full((L, 1), j, jnp.int32), _GDN,
                    slice_sizes=(1,),
                    mode=lax.GatherScatterMode.PROMISE_IN_BOUNDS)
                rows[base + j] = rows[base + j] * v
            return 0

        lax.fori_loop(0, BLKE // L, sc_, 0)
        _s3.__exit__(None, None, None)
        _s4 = jax.named_scope("pb_scatter"); _s4.__enter__()

        ds_ = [pltpu.async_copy(rows.at[pl.ds(j * SUB, SUB)],
                                out_sp.at[sb.at[j]], sem3, add=True)
               for j in range(RPB)]
        for d in ds_:
            d.wait()
        _s4.__exit__(None, None, None)
        return 0

    lax.fori_loop(0, BBLK, pb_loop, 0)
    plsc.subcore_barrier()

    with jax.named_scope("ph_writeout"):
        o0 = sid * OSL
        pltpu.sync_copy(out_sp.at[pl.ds(o0, OSL)], rows.at[pl.ds(0, OSL)])
        pltpu.sync_copy(rows.at[pl.ds(0, OSL)],
                        part.at[cid, pl.ds(o0, OSL)])


@functools.cache
def _sc_kernel():
    mesh = plsc.VectorSubcoreMesh(core_axis_name="c", subcore_axis_name="s")
    return pl.kernel(
        _sc_body,
        out_type=jax.ShapeDtypeStruct((NC, N, H1), jnp.float32),
        mesh=mesh,
        compiler_params=pltpu.CompilerParams(use_tc_tiling_on_sc=False),
        scratch_types=[
            pltpu.VMEM_SHARED((R * N,), jnp.float32),
            pltpu.VMEM_SHARED((N, H1), jnp.float32),
            pltpu.VMEM((RPB, SUB), jnp.int32),
            pltpu.VMEM((RPB, SUB), jnp.int32),
            pltpu.VMEM((RPB, SUB), jnp.int32),
            pltpu.VMEM((RPB, SUB), jnp.int32),
            pltpu.VMEM((RPB, SUB), jnp.int32),
            pltpu.VMEM((BLKE,), jnp.float32),
            pltpu.VMEM((SUB,), jnp.float32),
            pltpu.VMEM((BLKE, H1), jnp.float32),
            pltpu.SemaphoreType.DMA,
            pltpu.SemaphoreType.DMA,
            pltpu.SemaphoreType.DMA,
        ],
    )


def kernel(nodes, triples, weights, bias):
    s = triples[:, 0].reshape(EROWS, SUB)
    p = triples[:, 1].reshape(EROWS, SUB)
    o = triples[:, 2].reshape(EROWS, SUB)
    w_all = jnp.transpose(weights, (1, 0, 2)).reshape(H0, R * H1)
    nw2 = _mm(nodes, w_all).reshape(N * R, H1)
    part = _sc_kernel()(s, p, o, nw2)
    bias_t = jnp.tile(bias, H0 // H1).reshape(1, 128)
    out2 = _comb(part.reshape(NC, (N * H1) // 128, 128), bias_t)
    return out2.reshape(N, H1)


# speedup vs baseline: 1.5155x; 1.0385x over previous
import functools

import jax
import jax.numpy as jnp
from jax import lax
from jax.experimental import pallas as pl
from jax.experimental.pallas import tpu as pltpu
from jax.experimental.pallas import tpu_sc as plsc

N = 10000
R = 16
E = 320000
H0 = 128
H1 = 16

NC = 2
NS = 16
L = 16

SUB = 80
EROWS = E // SUB
RPB = 25
BLKE = RPB * SUB
AROWS = EROWS // NS
ABLK = AROWS // RPB
BROWS = EROWS // (NC * NS)
BBLK = BROWS // RPB
CSL = (R * N) // NS
OSL = N // NS

_GDN = lax.GatherDimensionNumbers(
    offset_dims=(), collapsed_slice_dims=(0,), start_index_map=(0,))


def _mm_body(n_ref, w_ref, o_ref):
    o_ref[...] = jnp.dot(n_ref[...], w_ref[...],
                         preferred_element_type=jnp.float32)


_BM = 2000
_mm = pl.pallas_call(
    _mm_body,
    grid=(N // _BM,),
    in_specs=[
        pl.BlockSpec((_BM, H0), lambda i: (i, 0)),
        pl.BlockSpec((H0, R * H1), lambda i: (0, 0)),
    ],
    out_specs=pl.BlockSpec((_BM, R * H1), lambda i: (i, 0)),
    out_shape=jax.ShapeDtypeStruct((N, R * H1), jnp.float32),
)


def _comb_body(p_ref, b_ref, o_ref):
    o_ref[...] = p_ref[0] + p_ref[1] + b_ref[...]


_comb = pl.pallas_call(
    _comb_body,
    in_specs=[
        pl.BlockSpec((NC, (N * H1) // 128, 128), lambda: (0, 0, 0)),
        pl.BlockSpec((1, 128), lambda: (0, 0)),
    ],
    out_specs=pl.BlockSpec(((N * H1) // 128, 128), lambda: (0, 0)),
    out_shape=jax.ShapeDtypeStruct(((N * H1) // 128, 128), jnp.float32),
)


def _sc_body(tt3, nw2, part,
             counts_sp, out_sp, pb, qb, sb, keyb, gidxb, valb, ones,
             rows, sem, sem2, sem3):
    cid = lax.axis_index("c")
    sid = lax.axis_index("s")

    with jax.named_scope("ph_init"):
        for k in range(SUB // L):
            ones[pl.ds(k * L, L)] = jnp.ones((L,), jnp.float32)

        def zf(i, _):
            valb[pl.ds(i * L, L)] = jnp.zeros((L,), jnp.float32)
            return 0

        lax.fori_loop(0, BLKE // L, zf, 0, unroll=8)

        def zr(i, _):
            rows[i] = jnp.zeros((L,), jnp.float32)
            return 0

        lax.fori_loop(0, OSL, zr, 0, unroll=8)

        for t in range(CSL // BLKE):
            pltpu.sync_copy(valb,
                            counts_sp.at[pl.ds(sid * CSL + t * BLKE, BLKE)])
        pltpu.sync_copy(rows.at[pl.ds(0, OSL)],
                        out_sp.at[pl.ds(sid * OSL, OSL)])
        plsc.subcore_barrier()

    arow0 = sid * AROWS

    def pa(b, _):
        r0 = arow0 + b * RPB
        d1 = pltpu.async_copy(tt3.at[1, pl.ds(r0, RPB)], pb, sem)
        d2 = pltpu.async_copy(tt3.at[0, pl.ds(r0, RPB)], sb, sem)
        d1.wait(); d2.wait()

        def mk(j, _):
            for k in range(SUB // L):
                sl = pl.ds(k * L, L)
                keyb[j, sl] = pb[j, sl] * N + sb[j, sl]
            return 0

        lax.fori_loop(0, RPB, mk, 0, unroll=5)

        dsc = [pltpu.async_copy(ones, counts_sp.at[keyb.at[j]], sem3,
                                add=True)
               for j in range(RPB)]
        for d in dsc:
            d.wait()
        return 0

    with jax.named_scope("ph_count"):
        lax.fori_loop(0, ABLK, pa, 0)
        plsc.subcore_barrier()

    brow0 = cid * (EROWS // NC) + sid * BROWS

    def pb_loop(g, _):
        r0 = brow0 + g * RPB
        _s1 = jax.named_scope("pb_stage"); _s1.__enter__()
        d1 = pltpu.async_copy(tt3.at[1, pl.ds(r0, RPB)], pb, sem)
        d2 = pltpu.async_copy(tt3.at[2, pl.ds(r0, RPB)], qb, sem)
        d3 = pltpu.async_copy(tt3.at[0, pl.ds(r0, RPB)], sb, sem)
        d1.wait(); d2.wait(); d3.wait()

        def mk(j, _):
            for k in range(SUB // L):
                sl = pl.ds(k * L, L)
                pv = pb[j, sl]
                keyb[j, sl] = pv * N + sb[j, sl]
                gidxb[j, sl] = qb[j, sl] * H1 + pv
            return 0

        lax.fori_loop(0, RPB, mk, 0, unroll=5)
        _s1.__exit__(None, None, None)
        _s2 = jax.named_scope("pb_gather"); _s2.__enter__()

        dc = [pltpu.async_copy(counts_sp.at[keyb.at[j]],
                               valb.at[pl.ds(j * SUB, SUB)], sem)
              for j in range(RPB)]
        dr = [pltpu.async_copy(nw2.at[gidxb.at[j]],
                               rows.at[pl.ds(j * SUB, SUB)], sem2)
              for j in range(RPB)]
        for d in dc:
            d.wait()

        def inv(i, _):
            sl = pl.ds(i * L, L)
            valb[sl] = 1.0 / valb[sl]
            return 0

        lax.fori_loop(0, BLKE // L, inv, 0, unroll=5)
        for d in dr:
            d.wait()
        _s2.__exit__(None, None, None)
        _s3 = jax.named_scope("pb_scale"); _s3.__enter__()

        def sc_(c, _):
            vch = valb[pl.ds(c * L, L)]
            base = c * L
            for j in range(L):
                v = lax.gather(
                    vch, jnp.full((L, 1), j, jnp.int32), _GDN,
                    slice_sizes=(1,),
                    mode=lax.GatherScatterMode.PROMISE_IN_BOUNDS)
                rows[base + j] = rows[base + j] * v
            return 0

        lax.fori_loop(0, BLKE // L, sc_, 0)
        _s3.__exit__(None, None, None)
        _s4 = jax.named_scope("pb_scatter"); _s4.__enter__()

        ds_ = [pltpu.async_copy(rows.at[pl.ds(j * SUB, SUB)],
                                out_sp.at[sb.at[j]], sem3, add=True)
               for j in range(RPB)]
        for d in ds_:
            d.wait()
        _s4.__exit__(None, None, None)
        return 0

    lax.fori_loop(0, BBLK, pb_loop, 0)
    plsc.subcore_barrier()

    with jax.named_scope("ph_writeout"):
        o0 = sid * OSL
        pltpu.sync_copy(out_sp.at[pl.ds(o0, OSL)], rows.at[pl.ds(0, OSL)])
        pltpu.sync_copy(rows.at[pl.ds(0, OSL)],
                        part.at[cid, pl.ds(o0, OSL)])


@functools.cache
def _sc_kernel():
    mesh = plsc.VectorSubcoreMesh(core_axis_name="c", subcore_axis_name="s")
    return pl.kernel(
        _sc_body,
        out_type=jax.ShapeDtypeStruct((NC, N, H1), jnp.float32),
        mesh=mesh,
        compiler_params=pltpu.CompilerParams(use_tc_tiling_on_sc=False),
        scratch_types=[
            pltpu.VMEM_SHARED((R * N,), jnp.float32),
            pltpu.VMEM_SHARED((N, H1), jnp.float32),
            pltpu.VMEM((RPB, SUB), jnp.int32),
            pltpu.VMEM((RPB, SUB), jnp.int32),
            pltpu.VMEM((RPB, SUB), jnp.int32),
            pltpu.VMEM((RPB, SUB), jnp.int32),
            pltpu.VMEM((RPB, SUB), jnp.int32),
            pltpu.VMEM((BLKE,), jnp.float32),
            pltpu.VMEM((SUB,), jnp.float32),
            pltpu.VMEM((BLKE, H1), jnp.float32),
            pltpu.SemaphoreType.DMA,
            pltpu.SemaphoreType.DMA,
            pltpu.SemaphoreType.DMA,
        ],
    )


def kernel(nodes, triples, weights, bias):
    tt3 = triples.T.reshape(3, EROWS, SUB)
    w_all = jnp.transpose(weights, (1, 0, 2)).reshape(H0, R * H1)
    nw2 = _mm(nodes, w_all).reshape(N * R, H1)
    part = _sc_kernel()(tt3, nw2)
    bias_t = jnp.tile(bias, H0 // H1).reshape(1, 128)
    out2 = _comb(part.reshape(NC, (N * H1) // 128, 128), bias_t)
    return out2.reshape(N, H1)


# speedup vs baseline: 1.6653x; 1.0989x over previous
import functools

import jax
import jax.numpy as jnp
from jax import lax
from jax.experimental import pallas as pl
from jax.experimental.pallas import tpu as pltpu
from jax.experimental.pallas import tpu_sc as plsc

N = 10000
R = 16
E = 320000
H0 = 128
H1 = 16

NC = 2
NS = 16
L = 16

SUB = 80
EROWS = E // SUB
RPB = 25
BLKE = RPB * SUB
AROWS = EROWS // NS
ABLK = AROWS // RPB
BROWS = EROWS // (NC * NS)
BBLK = BROWS // RPB
CSL = (R * N) // NS
OSL = N // NS

_GDN = lax.GatherDimensionNumbers(
    offset_dims=(), collapsed_slice_dims=(0,), start_index_map=(0,))


def _mm_body(n_ref, w_ref, o_ref):
    o_ref[...] = jnp.dot(n_ref[...], w_ref[...],
                         preferred_element_type=jnp.float32)


_BM = 2000
_mm = pl.pallas_call(
    _mm_body,
    grid=(N // _BM,),
    in_specs=[
        pl.BlockSpec((_BM, H0), lambda i: (i, 0)),
        pl.BlockSpec((H0, R * H1), lambda i: (0, 0)),
    ],
    out_specs=pl.BlockSpec((_BM, R * H1), lambda i: (i, 0)),
    out_shape=jax.ShapeDtypeStruct((N, R * H1), jnp.float32),
)


def _comb_body(p_ref, b_ref, o_ref):
    o_ref[...] = p_ref[0] + p_ref[1] + b_ref[...]


_comb = pl.pallas_call(
    _comb_body,
    in_specs=[
        pl.BlockSpec((NC, (N * H1) // 128, 128), lambda: (0, 0, 0)),
        pl.BlockSpec((1, 128), lambda: (0, 0)),
    ],
    out_specs=pl.BlockSpec(((N * H1) // 128, 128), lambda: (0, 0)),
    out_shape=jax.ShapeDtypeStruct(((N * H1) // 128, 128), jnp.float32),
)


def _sc_body(tt3, nw2, part,
             counts_sp, out_sp, pb, qb, sb, keyb, gidxb, valb, ones,
             rows, sem, sem2, sem3):
    cid = lax.axis_index("c")
    sid = lax.axis_index("s")

    with jax.named_scope("ph_init"):
        for k in range(SUB // L):
            ones[pl.ds(k * L, L)] = jnp.ones((L,), jnp.float32)

        def zf(i, _):
            valb[pl.ds(i * L, L)] = jnp.zeros((L,), jnp.float32)
            return 0

        lax.fori_loop(0, BLKE // L, zf, 0, unroll=8)

        def zr(i, _):
            rows[i] = jnp.zeros((L,), jnp.float32)
            return 0

        lax.fori_loop(0, OSL, zr, 0, unroll=8)

        for t in range(CSL // BLKE):
            pltpu.sync_copy(valb,
                            counts_sp.at[pl.ds(sid * CSL + t * BLKE, BLKE)])
        pltpu.sync_copy(rows.at[pl.ds(0, OSL)],
                        out_sp.at[pl.ds(sid * OSL, OSL)])
        plsc.subcore_barrier()

    arow0 = sid * AROWS

    def pa(b, _):
        r0 = arow0 + b * RPB
        d1 = pltpu.async_copy(tt3.at[1, pl.ds(r0, RPB)], pb, sem)
        d2 = pltpu.async_copy(tt3.at[0, pl.ds(r0, RPB)], sb, sem)
        d1.wait(); d2.wait()

        def mk(j, _):
            for k in range(SUB // L):
                sl = pl.ds(k * L, L)
                keyb[j, sl] = pb[j, sl] * N + sb[j, sl]
            return 0

        lax.fori_loop(0, RPB, mk, 0, unroll=5)

        dsc = [pltpu.async_copy(ones, counts_sp.at[keyb.at[j]], sem3,
                                add=True)
               for j in range(RPB)]
        for d in dsc:
            d.wait()
        return 0

    with jax.named_scope("ph_count"):
        lax.fori_loop(0, ABLK, pa, 0)
        plsc.subcore_barrier()

    brow0 = cid * (EROWS // NC) + sid * BROWS

    def pb_loop(g, _):
        r0 = brow0 + g * RPB
        _s1 = jax.named_scope("pb_stage"); _s1.__enter__()
        d1 = pltpu.async_copy(tt3.at[1, pl.ds(r0, RPB)], pb, sem)
        d2 = pltpu.async_copy(tt3.at[2, pl.ds(r0, RPB)], qb, sem)
        d3 = pltpu.async_copy(tt3.at[0, pl.ds(r0, RPB)], sb, sem)
        d1.wait(); d2.wait(); d3.wait()

        def mk(j, _):
            for k in range(SUB // L):
                sl = pl.ds(k * L, L)
                pv = pb[j, sl]
                ov = qb[j, sl]
                keyb[j, sl] = pv * N + sb[j, sl]
                gidxb[j, sl] = ((ov << 3) + ((ov >> 3) << 6)
                                + pv + ((pv >> 3) * 56))
            return 0

        lax.fori_loop(0, RPB, mk, 0, unroll=5)
        _s1.__exit__(None, None, None)
        _s2 = jax.named_scope("pb_gather"); _s2.__enter__()

        dc = [pltpu.async_copy(counts_sp.at[keyb.at[j]],
                               valb.at[pl.ds(j * SUB, SUB)], sem)
              for j in range(RPB)]
        dr = [pltpu.async_copy(nw2.at[gidxb.at[j]],
                               rows.at[pl.ds(j * SUB, SUB)], sem2)
              for j in range(RPB)]
        for d in dc:
            d.wait()

        def inv(i, _):
            sl = pl.ds(i * L, L)
            valb[sl] = 1.0 / valb[sl]
            return 0

        lax.fori_loop(0, BLKE // L, inv, 0, unroll=5)
        for d in dr:
            d.wait()
        _s2.__exit__(None, None, None)
        _s3 = jax.named_scope("pb_scale"); _s3.__enter__()

        def sc_(c, _):
            vch = valb[pl.ds(c * L, L)]
            base = c * L
            for j in range(L):
                v = lax.gather(
                    vch, jnp.full((L, 1), j, jnp.int32), _GDN,
                    slice_sizes=(1,),
                    mode=lax.GatherScatterMode.PROMISE_IN_BOUNDS)
                rows[base + j] = rows[base + j] * v
            return 0

        lax.fori_loop(0, BLKE // L, sc_, 0)
        _s3.__exit__(None, None, None)
        _s4 = jax.named_scope("pb_scatter"); _s4.__enter__()

        ds_ = [pltpu.async_copy(rows.at[pl.ds(j * SUB, SUB)],
                                out_sp.at[sb.at[j]], sem3, add=True)
               for j in range(RPB)]
        for d in ds_:
            d.wait()
        _s4.__exit__(None, None, None)
        return 0

    lax.fori_loop(0, BBLK, pb_loop, 0)
    plsc.subcore_barrier()

    with jax.named_scope("ph_writeout"):
        o0 = sid * OSL
        pltpu.sync_copy(out_sp.at[pl.ds(o0, OSL)], rows.at[pl.ds(0, OSL)])
        pltpu.sync_copy(rows.at[pl.ds(0, OSL)],
                        part.at[cid, pl.ds(o0, OSL)])


@functools.cache
def _sc_kernel():
    mesh = plsc.VectorSubcoreMesh(core_axis_name="c", subcore_axis_name="s")
    return pl.kernel(
        _sc_body,
        out_type=jax.ShapeDtypeStruct((NC, N, H1), jnp.float32),
        mesh=mesh,
        compiler_params=pltpu.CompilerParams(use_tc_tiling_on_sc=False),
        scratch_types=[
            pltpu.VMEM_SHARED((R * N,), jnp.float32),
            pltpu.VMEM_SHARED((N, H1), jnp.float32),
            pltpu.VMEM((RPB, SUB), jnp.int32),
            pltpu.VMEM((RPB, SUB), jnp.int32),
            pltpu.VMEM((RPB, SUB), jnp.int32),
            pltpu.VMEM((RPB, SUB), jnp.int32),
            pltpu.VMEM((RPB, SUB), jnp.int32),
            pltpu.VMEM((BLKE,), jnp.float32),
            pltpu.VMEM((SUB,), jnp.float32),
            pltpu.VMEM((BLKE, H1), jnp.float32),
            pltpu.SemaphoreType.DMA,
            pltpu.SemaphoreType.DMA,
            pltpu.SemaphoreType.DMA,
        ],
    )


def kernel(nodes, triples, weights, bias):
    tt3 = triples.T.reshape(3, EROWS, SUB)
    w_all = jnp.transpose(weights, (1, 0, 2)).reshape(H0, R * H1)
    nw_all = _mm(nodes, w_all)
    nw2 = (nw_all.reshape(N // 8, 8, 2, 128)
           .transpose(0, 2, 1, 3).reshape(N * R, H1))
    part = _sc_kernel()(tt3, nw2)
    bias_t = jnp.tile(bias, H0 // H1).reshape(1, 128)
    out2 = _comb(part.reshape(NC, (N * H1) // 128, 128), bias_t)
    return out2.reshape(N, H1)


# speedup vs baseline: 1.8552x; 1.1141x over previous
import functools

import jax
import jax.numpy as jnp
from jax import lax
from jax.experimental import pallas as pl
from jax.experimental.pallas import tpu as pltpu
from jax.experimental.pallas import tpu_sc as plsc

N = 10000
R = 16
E = 320000
H0 = 128
H1 = 16

NC = 2
NS = 16
L = 16

SUB = 80
EROWS = E // SUB
RPB = 25
BLKE = RPB * SUB
AROWS = EROWS // NS
ABLK = AROWS // RPB
BROWS = EROWS // (NC * NS)
BBLK = BROWS // RPB
CSL = (R * N) // NS
OSL = N // NS

_GDN = lax.GatherDimensionNumbers(
    offset_dims=(), collapsed_slice_dims=(0,), start_index_map=(0,))


def _mm_body(n_ref, w_ref, o_ref):
    o_ref[...] = jnp.dot(n_ref[...], w_ref[...],
                         preferred_element_type=jnp.float32)


_BM = 2000
_mm = pl.pallas_call(
    _mm_body,
    grid=(N // _BM,),
    in_specs=[
        pl.BlockSpec((_BM, H0), lambda i: (i, 0)),
        pl.BlockSpec((H0, R * H1), lambda i: (0, 0)),
    ],
    out_specs=pl.BlockSpec((_BM, R * H1), lambda i: (i, 0)),
    out_shape=jax.ShapeDtypeStruct((N, R * H1), jnp.float32),
)


def _comb_body(p_ref, b_ref, o_ref):
    o_ref[...] = p_ref[0] + p_ref[1] + b_ref[...]


_comb = pl.pallas_call(
    _comb_body,
    in_specs=[
        pl.BlockSpec((NC, (N * H1) // 128, 128), lambda: (0, 0, 0)),
        pl.BlockSpec((1, 128), lambda: (0, 0)),
    ],
    out_specs=pl.BlockSpec(((N * H1) // 128, 128), lambda: (0, 0)),
    out_shape=jax.ShapeDtypeStruct(((N * H1) // 128, 128), jnp.float32),
)


def _sc_body(tt3, nw2, part,
             counts_sp, out_sp,
             pb0, qb0, sb0, keyb0, gidxb0,
             pb1, qb1, sb1, keyb1, gidxb1,
             valb, ones, rows0, rows1,
             sem_ld, sem_cnt, sem_row, sem_sc):
    cid = lax.axis_index("c")
    sid = lax.axis_index("s")
    pbx, qbx, sbx = (pb0, pb1), (qb0, qb1), (sb0, sb1)
    keybx, gidxbx, rowsx = (keyb0, keyb1), (gidxb0, gidxb1), (rows0, rows1)

    with jax.named_scope("ph_init"):
        for k in range(SUB // L):
            ones[pl.ds(k * L, L)] = jnp.ones((L,), jnp.float32)

        def zf(i, _):
            valb[pl.ds(i * L, L)] = jnp.zeros((L,), jnp.float32)
            return 0

        lax.fori_loop(0, BLKE // L, zf, 0, unroll=4)

        def zr(i, _):
            rows0[i] = jnp.zeros((L,), jnp.float32)
            return 0

        lax.fori_loop(0, OSL, zr, 0, unroll=4)

        for t in range(CSL // BLKE):
            pltpu.sync_copy(valb,
                            counts_sp.at[pl.ds(sid * CSL + t * BLKE, BLKE)])
        pltpu.sync_copy(rows0.at[pl.ds(0, OSL)],
                        out_sp.at[pl.ds(sid * OSL, OSL)])
        plsc.subcore_barrier()

    with jax.named_scope("ph_count"):
        arow0 = sid * AROWS

        def akeys(par):
            pbr, sbr, keyr = pbx[par], sbx[par], keybx[par]

            def mk(j, _):
                for k in range(SUB // L):
                    sl = pl.ds(k * L, L)
                    keyr[j, sl] = pbr[j, sl] * N + sbr[j, sl]
                return 0

            lax.fori_loop(0, RPB, mk, 0)

        def afire_loads(b):
            r0 = arow0 + b * RPB
            return [
                pltpu.async_copy(tt3.at[1, pl.ds(r0, RPB)], pbx[b % 2],
                                 sem_ld),
                pltpu.async_copy(tt3.at[0, pl.ds(r0, RPB)], sbx[b % 2],
                                 sem_ld),
            ]

        dl = afire_loads(0)
        dsc_prev2, dsc_prev1 = [], []
        for b in range(ABLK):
            par = b % 2
            for d in dl:
                d.wait()
            if b + 1 < ABLK:
                dl = afire_loads(b + 1)
            for d in dsc_prev2:
                d.wait()
            akeys(par)
            dsc_prev2 = dsc_prev1
            dsc_prev1 = [
                pltpu.async_copy(ones, counts_sp.at[keybx[par].at[j]],
                                 sem_sc, add=True)
                for j in range(RPB)
            ]
        for d in dsc_prev2 + dsc_prev1:
            d.wait()
        plsc.subcore_barrier()

    with jax.named_scope("ph_main"):
        brow0 = cid * (EROWS // NC) + sid * BROWS

        def bfire_loads(g):
            r0 = brow0 + g * RPB
            par = g % 2
            return [
                pltpu.async_copy(tt3.at[1, pl.ds(r0, RPB)], pbx[par],
                                 sem_ld),
                pltpu.async_copy(tt3.at[2, pl.ds(r0, RPB)], qbx[par],
                                 sem_ld),
                pltpu.async_copy(tt3.at[0, pl.ds(r0, RPB)], sbx[par],
                                 sem_ld),
            ]

        def bkeys(par):
            pbr, qbr, sbr = pbx[par], qbx[par], sbx[par]
            keyr, gidxr = keybx[par], gidxbx[par]

            def mk(j, _):
                for k in range(SUB // L):
                    sl = pl.ds(k * L, L)
                    pv = pbr[j, sl]
                    ov = qbr[j, sl]
                    keyr[j, sl] = pv * N + sbr[j, sl]
                    gidxr[j, sl] = ((ov << 3) + ((ov >> 3) << 6)
                                    + pv + ((pv >> 3) * 56))
                return 0

            lax.fori_loop(0, RPB, mk, 0)

        def bscale(par):
            rr = rowsx[par]

            def sc_(c, _):
                vch = valb[pl.ds(c * L, L)]
                base = c * L
                for j in range(L):
                    v = lax.gather(
                        vch, jnp.full((L, 1), j, jnp.int32), _GDN,
                        slice_sizes=(1,),
                        mode=lax.GatherScatterMode.PROMISE_IN_BOUNDS)
                    rr[base + j] = rr[base + j] * v
                return 0

            lax.fori_loop(0, BLKE // L, sc_, 0)

        dl = bfire_loads(0)
        dscat = []
        for g in range(BBLK):
            par = g % 2
            for d in dl:
                d.wait()
            bkeys(par)
            for d in dscat:
                d.wait()
            if g + 1 < BBLK:
                dl = bfire_loads(g + 1)
            dcnt = [
                pltpu.async_copy(counts_sp.at[keybx[par].at[j]],
                                 valb.at[pl.ds(j * SUB, SUB)], sem_cnt)
                for j in range(RPB)
            ]
            drow = [
                pltpu.async_copy(nw2.at[gidxbx[par].at[j]],
                                 rowsx[par].at[pl.ds(j * SUB, SUB)],
                                 sem_row)
                for j in range(RPB)
            ]
            for d in dcnt:
                d.wait()

            def inv(i, _):
                sl = pl.ds(i * L, L)
                valb[sl] = 1.0 / valb[sl]
                return 0

            lax.fori_loop(0, BLKE // L, inv, 0, unroll=2)
            for d in drow:
                d.wait()
            bscale(par)
            dscat = [
                pltpu.async_copy(rowsx[par].at[pl.ds(j * SUB, SUB)],
                                 out_sp.at[sbx[par].at[j]], sem_sc,
                                 add=True)
                for j in range(RPB)
            ]
        for d in dscat:
            d.wait()
        plsc.subcore_barrier()

    with jax.named_scope("ph_writeout"):
        o0 = sid * OSL
        pltpu.sync_copy(out_sp.at[pl.ds(o0, OSL)], rows0.at[pl.ds(0, OSL)])
        pltpu.sync_copy(rows0.at[pl.ds(0, OSL)],
                        part.at[cid, pl.ds(o0, OSL)])


@functools.cache
def _sc_kernel():
    mesh = plsc.VectorSubcoreMesh(core_axis_name="c", subcore_axis_name="s")
    ibuf = pltpu.VMEM((RPB, SUB), jnp.int32)
    return pl.kernel(
        _sc_body,
        out_type=jax.ShapeDtypeStruct((NC, N, H1), jnp.float32),
        mesh=mesh,
        compiler_params=pltpu.CompilerParams(use_tc_tiling_on_sc=False),
        scratch_types=[
            pltpu.VMEM_SHARED((R * N,), jnp.float32),
            pltpu.VMEM_SHARED((N, H1), jnp.float32),
            ibuf, ibuf, ibuf, ibuf, ibuf,
            ibuf, ibuf, ibuf, ibuf, ibuf,
            pltpu.VMEM((BLKE,), jnp.float32),
            pltpu.VMEM((SUB,), jnp.float32),
            pltpu.VMEM((BLKE, H1), jnp.float32),
            pltpu.VMEM((BLKE, H1), jnp.float32),
            pltpu.SemaphoreType.DMA,
            pltpu.SemaphoreType.DMA,
            pltpu.SemaphoreType.DMA,
            pltpu.SemaphoreType.DMA,
        ],
    )


def kernel(nodes, triples, weights, bias):
    tt3 = triples.T.reshape(3, EROWS, SUB)
    w_all = jnp.transpose(weights, (1, 0, 2)).reshape(H0, R * H1)
    nw_all = _mm(nodes, w_all)
    nw2 = (nw_all.reshape(N // 8, 8, 2, 128)
           .transpose(0, 2, 1, 3).reshape(N * R, H1))
    part = _sc_kernel()(tt3, nw2)
    bias_t = jnp.tile(bias, H0 // H1).reshape(1, 128)
    out2 = _comb(part.reshape(NC, (N * H1) // 128, 128), bias_t)
    return out2.reshape(N, H1)


# speedup vs baseline: 2.0177x; 1.0876x over previous
import functools

import jax
import jax.numpy as jnp
from jax import lax
from jax.experimental import pallas as pl
from jax.experimental.pallas import tpu as pltpu
from jax.experimental.pallas import tpu_sc as plsc

N = 10000
R = 16
E = 320000
H0 = 128
H1 = 16

NC = 2
NS = 16
L = 16

SUB = 80
EROWS = E // SUB
RPB = 25
BLKE = RPB * SUB
AROWS = EROWS // NS
ABLK = AROWS // RPB
BROWS = EROWS // (NC * NS)
BBLK = BROWS // RPB
CSL = (R * N) // NS
OSL = N // NS

_GDN = lax.GatherDimensionNumbers(
    offset_dims=(), collapsed_slice_dims=(0,), start_index_map=(0,))


def _mm_body(n_ref, w_ref, o_ref):
    o_ref[...] = jnp.dot(n_ref[...], w_ref[...],
                         preferred_element_type=jnp.float32)


_BM = 2000
_mm = pl.pallas_call(
    _mm_body,
    grid=(N // _BM,),
    in_specs=[
        pl.BlockSpec((_BM, H0), lambda i: (i, 0)),
        pl.BlockSpec((H0, R * H1), lambda i: (0, 0)),
    ],
    out_specs=pl.BlockSpec((_BM, R * H1), lambda i: (i, 0)),
    out_shape=jax.ShapeDtypeStruct((N, R * H1), jnp.float32),
)


def _comb_body(p_ref, b_ref, o_ref):
    o_ref[...] = p_ref[0] + p_ref[1] + b_ref[...]


_comb = pl.pallas_call(
    _comb_body,
    in_specs=[
        pl.BlockSpec((NC, (N * H1) // 128, 128), lambda: (0, 0, 0)),
        pl.BlockSpec((1, 128), lambda: (0, 0)),
    ],
    out_specs=pl.BlockSpec(((N * H1) // 128, 128), lambda: (0, 0)),
    out_shape=jax.ShapeDtypeStruct(((N * H1) // 128, 128), jnp.float32),
)


def _sc_body(tt3, nw2, part,
             counts_sp, out_sp,
             pb0, qb0, sb0, keyb0, gidxb0,
             pb1, qb1, sb1, keyb1, gidxb1,
             sidx0, sidx1, sidx2,
             valb, valb1, ones, rows0, rows1,
             sem_ld, sem_cnt, sem_row, sem_sc):
    cid = lax.axis_index("c")
    sid = lax.axis_index("s")
    pbx, qbx, sbx = (pb0, pb1), (qb0, qb1), (sb0, sb1)
    keybx, gidxbx = (keyb0, keyb1), (gidxb0, gidxb1)
    rowsx = (rows0, rows1)
    sidxx = (sidx0, sidx1, sidx2)
    valbx = (valb, valb1)

    with jax.named_scope("ph_init"):
        for k in range(SUB // L):
            ones[pl.ds(k * L, L)] = jnp.ones((L,), jnp.float32)

        def zf(i, _):
            valb[pl.ds(i * L, L)] = jnp.zeros((L,), jnp.float32)
            return 0

        lax.fori_loop(0, BLKE // L, zf, 0, unroll=4)

        def zr(i, _):
            rows0[i] = jnp.zeros((L,), jnp.float32)
            return 0

        lax.fori_loop(0, OSL, zr, 0, unroll=4)

        for t in range(CSL // BLKE):
            pltpu.sync_copy(valb,
                            counts_sp.at[pl.ds(sid * CSL + t * BLKE, BLKE)])
        pltpu.sync_copy(rows0.at[pl.ds(0, OSL)],
                        out_sp.at[pl.ds(sid * OSL, OSL)])
        plsc.subcore_barrier()

    with jax.named_scope("ph_count"):
        arow0 = sid * AROWS

        def akeys(par):
            pbr, sbr, keyr = pbx[par], sbx[par], keybx[par]

            def mk(j, _):
                for k in range(SUB // L):
                    sl = pl.ds(k * L, L)
                    keyr[j, sl] = pbr[j, sl] * N + sbr[j, sl]
                return 0

            lax.fori_loop(0, RPB, mk, 0)

        def afire_loads(b):
            r0 = arow0 + b * RPB
            return [
                pltpu.async_copy(tt3.at[1, pl.ds(r0, RPB)], pbx[b % 2],
                                 sem_ld),
                pltpu.async_copy(tt3.at[0, pl.ds(r0, RPB)], sbx[b % 2],
                                 sem_ld),
            ]

        dl = afire_loads(0)
        dsc_prev2, dsc_prev1 = [], []
        for b in range(ABLK):
            par = b % 2
            for d in dl:
                d.wait()
            if b + 1 < ABLK:
                dl = afire_loads(b + 1)
            for d in dsc_prev2:
                d.wait()
            akeys(par)
            dsc_prev2 = dsc_prev1
            dsc_prev1 = [
                pltpu.async_copy(ones, counts_sp.at[keybx[par].at[j]],
                                 sem_sc, add=True)
                for j in range(RPB)
            ]
        for d in dsc_prev2 + dsc_prev1:
            d.wait()
        plsc.subcore_barrier()

    with jax.named_scope("ph_main"):
        brow0 = cid * (EROWS // NC) + sid * BROWS

        def bfire_loads(g):
            r0 = brow0 + g * RPB
            par = g % 2
            return [
                pltpu.async_copy(tt3.at[1, pl.ds(r0, RPB)], pbx[par],
                                 sem_ld),
                pltpu.async_copy(tt3.at[2, pl.ds(r0, RPB)], qbx[par],
                                 sem_ld),
                pltpu.async_copy(tt3.at[0, pl.ds(r0, RPB)], sbx[par],
                                 sem_ld),
            ]

        def bkeys(g):
            par = g % 2
            pbr, qbr, sbr = pbx[par], qbx[par], sbx[par]
            keyr, gidxr, sidxr = keybx[par], gidxbx[par], sidxx[g % 3]

            def mk(j, _):
                for k in range(SUB // L):
                    sl = pl.ds(k * L, L)
                    pv = pbr[j, sl]
                    ov = qbr[j, sl]
                    sv = sbr[j, sl]
                    keyr[j, sl] = pv * N + sv
                    sidxr[j, sl] = sv
                    gidxr[j, sl] = ((ov << 3) + ((ov >> 3) << 6)
                                    + pv + ((pv >> 3) * 56))
                return 0

            lax.fori_loop(0, RPB, mk, 0)

        def fire_gathers(g):
            par = g % 2
            dcnt = [
                pltpu.async_copy(counts_sp.at[keybx[par].at[j]],
                                 valbx[par].at[pl.ds(j * SUB, SUB)],
                                 sem_cnt)
                for j in range(RPB)
            ]
            drow = [
                pltpu.async_copy(nw2.at[gidxbx[par].at[j]],
                                 rowsx[g % 2].at[pl.ds(j * SUB, SUB)],
                                 sem_row)
                for j in range(RPB)
            ]
            return dcnt, drow

        def binv(g):
            vr = valbx[g % 2]

            def inv(i, _):
                sl = pl.ds(i * L, L)
                vr[sl] = 1.0 / vr[sl]
                return 0

            lax.fori_loop(0, BLKE // L, inv, 0, unroll=2)

        def bscale(g):
            rr = rowsx[g % 2]
            vr = valbx[g % 2]

            def sc_(c, _):
                vch = vr[pl.ds(c * L, L)]
                base = c * L
                for j in range(L):
                    v = lax.gather(
                        vch, jnp.full((L, 1), j, jnp.int32), _GDN,
                        slice_sizes=(1,),
                        mode=lax.GatherScatterMode.PROMISE_IN_BOUNDS)
                    rr[base + j] = rr[base + j] * v
                return 0

            lax.fori_loop(0, BLKE // L, sc_, 0)

        def fire_scats(g):
            return [
                pltpu.async_copy(rowsx[g % 2].at[pl.ds(j * SUB, SUB)],
                                 out_sp.at[sidxx[g % 3].at[j]], sem_sc,
                                 add=True)
                for j in range(RPB)
            ]

        dl = bfire_loads(0)
        for d in dl:
            d.wait()
        bkeys(0)
        dg = {0: fire_gathers(0)}
        dl = bfire_loads(1)
        dscats = {}
        for g in range(BBLK):
            if g + 1 < BBLK:
                for d in dl:
                    d.wait()
                bkeys(g + 1)
                if g - 1 in dscats:
                    for d in dscats.pop(g - 1):
                        d.wait()
                if g + 2 < BBLK:
                    dl = bfire_loads(g + 2)
                dg[g + 1] = fire_gathers(g + 1)
            elif g - 1 in dscats:
                for d in dscats.pop(g - 1):
                    d.wait()
            dcnt, drow = dg.pop(g)
            for d in dcnt:
                d.wait()
            binv(g)
            for d in drow:
                d.wait()
            bscale(g)
            dscats[g] = fire_scats(g)
        for g in sorted(dscats):
            for d in dscats.pop(g):
                d.wait()
        plsc.subcore_barrier()

    with jax.named_scope("ph_writeout"):
        o0 = sid * OSL
        pltpu.sync_copy(out_sp.at[pl.ds(o0, OSL)], rows0.at[pl.ds(0, OSL)])
        pltpu.sync_copy(rows0.at[pl.ds(0, OSL)],
                        part.at[cid, pl.ds(o0, OSL)])


@functools.cache
def _sc_kernel():
    mesh = plsc.VectorSubcoreMesh(core_axis_name="c", subcore_axis_name="s")
    ibuf = pltpu.VMEM((RPB, SUB), jnp.int32)
    return pl.kernel(
        _sc_body,
        out_type=jax.ShapeDtypeStruct((NC, N, H1), jnp.float32),
        mesh=mesh,
        compiler_params=pltpu.CompilerParams(use_tc_tiling_on_sc=False),
        scratch_types=[
            pltpu.VMEM_SHARED((R * N,), jnp.float32),
            pltpu.VMEM_SHARED((N, H1), jnp.float32),
            ibuf, ibuf, ibuf, ibuf, ibuf,
            ibuf, ibuf, ibuf, ibuf, ibuf,
            ibuf, ibuf, ibuf,
            pltpu.VMEM((BLKE,), jnp.float32),
            pltpu.VMEM((BLKE,), jnp.float32),
            pltpu.VMEM((SUB,), jnp.float32),
            pltpu.VMEM((BLKE, H1), jnp.float32),
            pltpu.VMEM((BLKE, H1), jnp.float32),
            pltpu.SemaphoreType.DMA,
            pltpu.SemaphoreType.DMA,
            pltpu.SemaphoreType.DMA,
            pltpu.SemaphoreType.DMA,
        ],
    )


def kernel(nodes, triples, weights, bias):
    tt3 = triples.T.reshape(3, EROWS, SUB)
    w_all = jnp.transpose(weights, (1, 0, 2)).reshape(H0, R * H1)
    nw_all = _mm(nodes, w_all)
    nw2 = (nw_all.reshape(N // 8, 8, 2, 128)
           .transpose(0, 2, 1, 3).reshape(N * R, H1))
    part = _sc_kernel()(tt3, nw2)
    bias_t = jnp.tile(bias, H0 // H1).reshape(1, 128)
    out2 = _comb(part.reshape(NC, (N * H1) // 128, 128), bias_t)
    return out2.reshape(N, H1)
